# Initial kernel scaffold; baseline (speedup 1.0000x reference)
import functools

import jax
import jax.numpy as jnp
from jax import lax
from jax.experimental import pallas as pl
from jax.experimental.pallas import tpu as pltpu
from jax.experimental.pallas import tpu_sc as plsc

DIM = 64
CHUNK = 512


@functools.lru_cache(maxsize=None)
def _make_sc_gather(B, D, C):
    info = plsc.get_sparse_core_info()
    NC, NS = info.num_cores, info.num_subcores
    NW = NC * NS
    assert B % (NW * C) == 0
    b_per_w = B // NW
    chunks = b_per_w // C
    mesh = plsc.VectorSubcoreMesh(core_axis_name="c", subcore_axis_name="s")

    @functools.partial(
        pl.kernel,
        mesh=mesh,
        out_type=jax.ShapeDtypeStruct((B, D), jnp.float32),
        scratch_types=[
            pltpu.VMEM((C,), jnp.int32),
            pltpu.VMEM((C, D), jnp.float32),
            pltpu.SemaphoreType.DMA,
        ],
    )
    def gather_kernel(idx_hbm, table_hbm, out_hbm, idx_v, rows_v, sem):
        wid = lax.axis_index("s") * NC + lax.axis_index("c")
        base = wid * b_per_w

        def body(g, carry):
            off = base + g * C
            pltpu.sync_copy(idx_hbm.at[pl.ds(off, C)], idx_v)
            pltpu.async_copy(table_hbm.at[idx_v], rows_v, sem).wait()
            pltpu.sync_copy(rows_v, out_hbm.at[pl.ds(off, C)])
            return carry

        lax.fori_loop(0, chunks, body, 0)

    return gather_kernel


def kernel(x, embedding):
    Bm, T = x.shape
    B = Bm * T
    flat_idx = x.reshape(B).astype(jnp.int32)
    out = _make_sc_gather(B, DIM, CHUNK)(flat_idx, embedding)
    return out.reshape(Bm, T, DIM)


# speedup vs baseline: 3.5858x; 3.5858x over previous
import functools

import jax
import jax.numpy as jnp
from jax import lax
from jax.experimental import pallas as pl
from jax.experimental.pallas import tpu as pltpu
from jax.experimental.pallas import tpu_sc as plsc

DIM = 64
CHUNK = 512


@functools.lru_cache(maxsize=None)
def _make_sc_gather(B, D, C):
    info = plsc.get_sparse_core_info()
    NC, NS = info.num_cores, info.num_subcores
    NW = NC * NS
    assert B % (NW * C) == 0
    b_per_w = B // NW
    chunks = b_per_w // C
    mesh = plsc.VectorSubcoreMesh(core_axis_name="c", subcore_axis_name="s")

    @functools.partial(
        pl.kernel,
        mesh=mesh,
        compiler_params=pltpu.CompilerParams(use_tc_tiling_on_sc=False),
        out_type=jax.ShapeDtypeStruct((B, D), jnp.float32),
        scratch_types=[
            pltpu.VMEM((C,), jnp.int32),
            pltpu.VMEM((C, D), jnp.float32),
            pltpu.SemaphoreType.DMA,
        ],
    )
    def gather_kernel(idx_hbm, table_hbm, out_hbm, idx_v, rows_v, sem):
        wid = lax.axis_index("s") * NC + lax.axis_index("c")
        base = wid * b_per_w

        def body(g, carry):
            off = base + g * C
            pltpu.sync_copy(idx_hbm.at[pl.ds(off, C)], idx_v)
            pltpu.async_copy(table_hbm.at[idx_v], rows_v, sem).wait()
            pltpu.sync_copy(rows_v, out_hbm.at[pl.ds(off, C)])
            return carry

        lax.fori_loop(0, chunks, body, 0)

    return gather_kernel


def kernel(x, embedding):
    Bm, T = x.shape
    B = Bm * T
    flat_idx = x.reshape(B).astype(jnp.int32)
    out = _make_sc_gather(B, DIM, CHUNK)(flat_idx, embedding)
    return out.reshape(Bm, T, DIM)


# speedup vs baseline: 3.5972x; 1.0032x over previous
import functools

import jax
import jax.numpy as jnp
from jax import lax
from jax.experimental import pallas as pl
from jax.experimental.pallas import tpu as pltpu
from jax.experimental.pallas import tpu_sc as plsc

DIM = 64
CHUNK = 512


@functools.lru_cache(maxsize=None)
def _make_sc_gather(B, D, C):
    info = plsc.get_sparse_core_info()
    NC, NS = info.num_cores, info.num_subcores
    NW = NC * NS
    assert B % (NW * C) == 0
    b_per_w = B // NW
    chunks = b_per_w // C
    assert chunks % 2 == 0 and chunks >= 6
    mesh = plsc.VectorSubcoreMesh(core_axis_name="c", subcore_axis_name="s")

    @functools.partial(
        pl.kernel,
        mesh=mesh,
        compiler_params=pltpu.CompilerParams(use_tc_tiling_on_sc=False),
        out_type=jax.ShapeDtypeStruct((B, D), jnp.float32),
        scratch_types=[
            pltpu.VMEM((C,), jnp.int32),
            pltpu.VMEM((C,), jnp.int32),
            pltpu.VMEM((C, D), jnp.float32),
            pltpu.VMEM((C, D), jnp.float32),
            pltpu.SemaphoreType.DMA,
            pltpu.SemaphoreType.DMA,
            pltpu.SemaphoreType.DMA,
            pltpu.SemaphoreType.DMA,
            pltpu.SemaphoreType.DMA,
        ],
    )
    def gather_kernel(idx_hbm, table_hbm, out_hbm, idx0, idx1, rows0, rows1,
                      sem_g, sem_i0, sem_i1, sem_o0, sem_o1):
        wid = lax.axis_index("s") * NC + lax.axis_index("c")
        base = wid * b_per_w
        idx_v = (idx0, idx1)
        rows_v = (rows0, rows1)
        sem_i = (sem_i0, sem_i1)
        sem_o = (sem_o0, sem_o1)

        for b in (0, 1):
            pltpu.sync_copy(idx_hbm.at[pl.ds(base + b * C, C)], idx_v[b])
            pltpu.async_copy(table_hbm.at[idx_v[b]], rows_v[b], sem_g).wait()
            pltpu.async_copy(idx_hbm.at[pl.ds(base + (b + 2) * C, C)],
                             idx_v[b], sem_i[b])
            pltpu.async_copy(rows_v[b], out_hbm.at[pl.ds(base + b * C, C)],
                             sem_o[b])

        def body(p, carry):
            for b in (0, 1):
                g = 2 * p + b
                off = base + g * C
                pltpu.make_async_copy(
                    rows_v[b], out_hbm.at[pl.ds(off - 2 * C, C)], sem_o[b]
                ).wait()
                pltpu.make_async_copy(
                    idx_hbm.at[pl.ds(off, C)], idx_v[b], sem_i[b]
                ).wait()
                pltpu.async_copy(table_hbm.at[idx_v[b]], rows_v[b],
                                 sem_g).wait()
                pltpu.async_copy(idx_hbm.at[pl.ds(off + 2 * C, C)],
                                 idx_v[b], sem_i[b])
                pltpu.async_copy(rows_v[b], out_hbm.at[pl.ds(off, C)],
                                 sem_o[b])
            return carry

        lax.fori_loop(1, chunks // 2 - 1, body, 0)

        for b in (0, 1):
            g = chunks - 2 + b
            off = base + g * C
            pltpu.make_async_copy(
                rows_v[b], out_hbm.at[pl.ds(off - 2 * C, C)], sem_o[b]
            ).wait()
            pltpu.make_async_copy(
                idx_hbm.at[pl.ds(off, C)], idx_v[b], sem_i[b]
            ).wait()
            pltpu.async_copy(table_hbm.at[idx_v[b]], rows_v[b], sem_g).wait()
            pltpu.async_copy(rows_v[b], out_hbm.at[pl.ds(off, C)], sem_o[b])
        for b in (0, 1):
            off = base + (chunks - 2 + b) * C
            pltpu.make_async_copy(
                rows_v[b], out_hbm.at[pl.ds(off, C)], sem_o[b]
            ).wait()

    return gather_kernel


def kernel(x, embedding):
    Bm, T = x.shape
    B = Bm * T
    flat_idx = x.reshape(B).astype(jnp.int32)
    out = _make_sc_gather(B, DIM, CHUNK)(flat_idx, embedding)
    return out.reshape(Bm, T, DIM)
